# baseline (speedup 1.0000x reference)
import functools

import numpy as np

import jax
import jax.numpy as jnp
from jax import lax
from jax.experimental import pallas as pl
from jax.experimental.pallas import tpu as pltpu
from jax.experimental.pallas import tpu_sc as plsc

EMB = 128
HEMB = 64
L = 5
G = 64
ALPHA = 0.2
THETA = 0.2

NP_PAD = 10240
CHUNK = 128
KCH = 8
BLK = CHUNK * KCH
DEA = 16


def _edge_pass_kernel(nbp, nbv, ebp):
    NPn = NP_PAD
    SL = NPn // 16
    mesh = plsc.VectorSubcoreMesh(core_axis_name="c", subcore_axis_name="s")

    @functools.partial(
        pl.kernel,
        out_type=(
            jax.ShapeDtypeStruct((2, 2, NPn, HEMB), jnp.float32),
            jax.ShapeDtypeStruct((2, NPn, DEA), jnp.float32),
            jax.ShapeDtypeStruct((2, 16, NPn), jnp.float32),
        ),
        mesh=mesh,
        compiler_params=pltpu.CompilerParams(
            needs_layout_passes=False, use_tc_tiling_on_sc=False),
        scratch_types=(
            pltpu.VMEM((NPn,), jnp.float32),
            pltpu.VMEM((NPn,), jnp.float32),
            pltpu.VMEM((NPn,), jnp.float32),
            pltpu.VMEM((KCH, CHUNK), jnp.int32),
            pltpu.VMEM((KCH, CHUNK), jnp.int32),
            pltpu.VMEM((KCH, CHUNK), jnp.int32),
            pltpu.VMEM((KCH, CHUNK), jnp.float32),
            pltpu.VMEM((KCH, CHUNK), jnp.float32),
            pltpu.VMEM((CHUNK, HEMB), jnp.bfloat16),
            pltpu.VMEM((CHUNK, HEMB), jnp.bfloat16),
            pltpu.VMEM((CHUNK, HEMB), jnp.float32),
            pltpu.VMEM((CHUNK, HEMB), jnp.float32),
            pltpu.VMEM((BLK, DEA), jnp.float32),
            pltpu.VMEM_SHARED((NPn, HEMB), jnp.float32),
            pltpu.VMEM_SHARED((NPn, DEA), jnp.float32),
            pltpu.SemaphoreType.DMA,
            pltpu.SemaphoreType.DMA,
            pltpu.SemaphoreType.DMA,
            pltpu.SemaphoreType.DMA,
            pltpu.SemaphoreType.DMA,
        ),
    )
    def edge_pass(src_hbm, dst_hbm, e3_hbm, ea_hbm, h2_hbm, a1_hbm, a2_hbm,
                  u_out, ea_out, den_out,
                  a1_v, a2_v, den_v, src_blk, dst_blk, gidx_blk, e3_blk,
                  ex_blk, rows_a, rows_b, sb_a, sb_b, ea_blk,
                  u_sh, easum_sh,
                  gsem_a, gsem_b, ssem_a, ssem_b, easem):
        c = lax.axis_index("c")
        s = lax.axis_index("s")
        rows = (rows_a, rows_b)
        sbufs = (sb_a, sb_b)
        gsem = (gsem_a, gsem_b)
        ssem = (ssem_a, ssem_b)

        z16 = jnp.zeros((16,), jnp.float32)

        def _graph(gph, _):
            nblocks = jnp.where(gph == 0, nbp, nbv)
            ebase = jnp.where(gph == 0, 0, ebp)
            aoff = pl.multiple_of(gph * NPn, NPn)
            srow = gph * (2 * NPn) + jnp.int32(c) * NPn

            pltpu.sync_copy(a1_hbm.at[pl.ds(aoff, NPn)], a1_v)
            pltpu.sync_copy(a2_hbm.at[pl.ds(aoff, NPn)], a2_v)

            def _zero_den(i, _):
                den_v[pl.ds(i * 16, 16)] = z16
                return 0
            lax.fori_loop(0, NPn // 16, _zero_den, 0)

            def _zero_rows(i, _):
                sb_a[i // 4, pl.ds((i % 4) * 16, 16)] = z16
                return 0
            lax.fori_loop(0, CHUNK * (HEMB // 16), _zero_rows, 0)

            def _zero_ea(i, _):
                ea_blk[i, :] = z16
                return 0
            lax.fori_loop(0, CHUNK, _zero_ea, 0)

            for k in range(SL // CHUNK):
                off = s * SL + k * CHUNK
                pltpu.sync_copy(sb_a, u_sh.at[pl.ds(off, CHUNK), :])
                pltpu.sync_copy(ea_blk.at[pl.ds(0, CHUNK), :],
                                easum_sh.at[pl.ds(off, CHUNK), :])
            plsc.subcore_barrier()

            def _block(b, _):
                blk0 = pl.multiple_of(ebase + (s * nblocks + b) * KCH, KCH)
                pltpu.sync_copy(src_hbm.at[pl.ds(blk0, KCH), :], src_blk)
                pltpu.sync_copy(dst_hbm.at[pl.ds(blk0, KCH), :], dst_blk)
                pltpu.sync_copy(e3_hbm.at[pl.ds(blk0, KCH), :], e3_blk)
                pltpu.sync_copy(ea_hbm.at[pl.ds(blk0 * CHUNK, BLK), :], ea_blk)

                gdesc = [None] * KCH
                sdesc = [None] * KCH
                eadesc = [None] * KCH

                def _score(k):
                    for g in range(CHUNK // 16):
                        sl16 = pl.ds(g * 16, 16)
                        s16 = src_blk[k, sl16]
                        d16 = dst_blk[k, sl16]
                        a1g = plsc.load_gather(a1_v, [s16])
                        a2g = plsc.load_gather(a2_v, [d16])
                        sc = a1g + a2g + e3_blk[k, sl16]
                        sc = jnp.where(sc >= 0.0, sc, 0.2 * sc)
                        ex = jnp.exp(sc)
                        ex_blk[k, sl16] = ex
                        plsc.addupdate_scatter(den_v, [d16], ex)
                        gidx_blk[k, sl16] = s16 + srow

                def _scale_scatter(k):
                    gdesc[k].wait()
                    if k >= 2:
                        sdesc[k - 2].wait()
                    rbuf = rows[k % 2]
                    sbuf = sbufs[k % 2]
                    for g in range(CHUNK // 16):
                        ex16 = ex_blk[k, pl.ds(g * 16, 16)]
                        for i in range(16):
                            e = g * 16 + i
                            exs = ex16[i]
                            for j in range(HEMB // 32):
                                m32 = rbuf[e, pl.ds(j * 32, 32)]
                                pa, pb = plsc.unpack(
                                    m32, format=plsc.PackFormat.INTERLEAVED,
                                    preferred_element_type=jnp.float32)
                                sbuf[e, pl.ds(j * 32, 16)] = pa * exs
                                sbuf[e, pl.ds(j * 32 + 16, 16)] = pb * exs
                            eo = k * CHUNK + e
                            ea_blk[eo, :] = ea_blk[eo, :] * exs
                    sdesc[k] = pltpu.async_copy(
                        sbuf, u_sh.at[dst_blk.at[k]], ssem[k % 2], add=True)
                    eadesc[k] = pltpu.async_copy(
                        ea_blk.at[pl.ds(k * CHUNK, CHUNK), :],
                        easum_sh.at[dst_blk.at[k]], easem, add=True)

                for k in range(KCH):
                    _score(k)
                    if k >= 2:
                        _scale_scatter(k - 2)
                    gdesc[k] = pltpu.async_copy(
                        h2_hbm.at[gidx_blk.at[k]], rows[k % 2], gsem[k % 2])
                for k in range(KCH - 2, KCH):
                    _scale_scatter(k)
                for k in range(KCH - 2, KCH):
                    sdesc[k].wait()
                for k in range(KCH):
                    eadesc[k].wait()
                return 0
            lax.fori_loop(0, nblocks, _block, 0)

            @pl.when(c == 0)
            def _():
                pltpu.sync_copy(den_v, den_out.at[gph, s])
            plsc.subcore_barrier()

            for k in range(SL // CHUNK):
                off = s * SL + k * CHUNK
                pltpu.sync_copy(u_sh.at[pl.ds(off, CHUNK), :],
                                u_out.at[gph, c, pl.ds(off, CHUNK), :])

                @pl.when(c == 0)
                def _():
                    pltpu.sync_copy(easum_sh.at[pl.ds(off, CHUNK), :],
                                    ea_out.at[gph, pl.ds(off, CHUNK), :])
            plsc.subcore_barrier()
            return 0

        lax.fori_loop(0, 2, _graph, 0)

    return edge_pass


BR = 512


def _dense_layer(beta):
    NPn = NP_PAD
    grid = (NPn // BR,)

    def body(u0, u1, eas, den16, h0, we, wl, aw, h_out, al_out):
        U = jnp.concatenate([u0[...], u1[...]], axis=1)
        den = jnp.sum(den16[...], axis=0)
        agg = (U + eas[...] @ we[...]) / (den + 1e-16)[:, None]
        sblk = (1.0 - ALPHA) * agg + ALPHA * h0[...]
        h = jax.nn.relu((1.0 - beta) * sblk + beta * (sblk @ wl[...]))
        h_out[...] = h
        al_out[...] = h @ aw[...]

    return pl.pallas_call(
        body,
        grid=grid,
        in_specs=[
            pl.BlockSpec((BR, HEMB), lambda i: (i, 0)),
            pl.BlockSpec((BR, HEMB), lambda i: (i, 0)),
            pl.BlockSpec((BR, DEA), lambda i: (i, 0)),
            pl.BlockSpec((16, BR), lambda i: (0, i)),
            pl.BlockSpec((BR, EMB), lambda i: (i, 0)),
            pl.BlockSpec((DEA, EMB), lambda i: (0, 0)),
            pl.BlockSpec((EMB, EMB), lambda i: (0, 0)),
            pl.BlockSpec((EMB, 2), lambda i: (0, 0)),
        ],
        out_specs=[
            pl.BlockSpec((BR, EMB), lambda i: (i, 0)),
            pl.BlockSpec((BR, 2), lambda i: (i, 0)),
        ],
        out_shape=[
            jax.ShapeDtypeStruct((NPn, EMB), jnp.float32),
            jax.ShapeDtypeStruct((NPn, 2), jnp.float32),
        ],
    )


def _heads_kernel():
    N = 10000

    def body(hp, hv, batch, wg_p, wv_p, wg_v, wv_v, va_in, aw1, aw2,
             fw1, fb1, fw2, fb2, fw3, fb3, wih, whh, bih, bhh,
             cw1, cb1, cw2, cb2, cw3, cb3, hid, logits_out, h_out):
        bi = jax.lax.broadcasted_iota(jnp.int32, (G, N), 0)
        B = (bi == batch[...]).astype(jnp.float32)

        def gap(x, wg, wv):
            t = jnp.tanh(x[...] @ wg[...])
            scgT = jax.lax.dot_general(wv[...], t,
                                       (((1,), (1,)), ((), ())))
            bs = jnp.where(B > 0, scgT, -1e30)
            m = jnp.max(bs, axis=1, keepdims=True)
            m = jnp.where(m < -9e29, 0.0, m)
            exn = jnp.where(B > 0, jnp.exp(scgT - m), 0.0)
            den = jnp.sum(exn, axis=1, keepdims=True)
            return (exn / (den + 1e-16)) @ x[...]

        pg = gap(hp, wg_p, wv_p)
        vg = gap(hv, wg_v, wv_v)
        va = jax.nn.relu(va_in[...] @ aw1[...]) @ aw2[...]
        fusion = jnp.concatenate([pg, vg, va], axis=-1)
        fe = jax.nn.relu(fusion @ fw1[...] + fb1[...])
        fe = jax.nn.relu(fe @ fw2[...] + fb2[...])
        fe = fe @ fw3[...] + fb3[...]
        gi = fe @ wih[...] + bih[...]
        gh = hid[...] @ whh[...] + bhh[...]
        ir, iz, inn = jnp.split(gi, 3, axis=-1)
        hr, hz, hn = jnp.split(gh, 3, axis=-1)
        r = jax.nn.sigmoid(ir + hr)
        z = jax.nn.sigmoid(iz + hz)
        n = jnp.tanh(inn + r * hn)
        h = (1.0 - z) * n + z * hid[...]
        lg = jax.nn.relu(h @ cw1[...] + cb1[...])
        lg = jax.nn.relu(lg @ cw2[...] + cb2[...])
        logits_out[...] = lg @ cw3[...] + cb3[...]
        h_out[...] = h

    return pl.pallas_call(
        body,
        out_shape=[
            jax.ShapeDtypeStruct((G, 100), jnp.float32),
            jax.ShapeDtypeStruct((G, EMB), jnp.float32),
        ],
    )


def _prep_edges(ei, ea_raw, p):
    E = ei.shape[1]
    EPG = 16 * BLK
    EP = ((E + EPG - 1) // EPG) * EPG
    src = jnp.concatenate([ei[0], jnp.zeros((EP - E,), ei.dtype)]).astype(jnp.int32)
    dst = jnp.concatenate([ei[1], jnp.zeros((EP - E,), ei.dtype)]).astype(jnp.int32)
    De = ea_raw.shape[1]
    ea = jnp.pad(ea_raw, ((0, EP - E), (0, DEA - De)))
    We = jnp.pad(p['We'], ((0, 0), (0, DEA - De), (0, 0)))
    c3 = jnp.einsum('lde,le->ld', We, p['a3'])
    e3 = ea @ c3.T
    e3 = jnp.where((jnp.arange(EP) < E)[:, None], e3, -1e30)
    return src, dst, e3, ea, We, EP


def _inproj_kernel(xp, win, aw0):
    def _inproj(xr, win_r, aw, h_out, al_out):
        h = jax.nn.relu(xr[...] @ win_r[...])
        h_out[...] = h
        al_out[...] = h @ aw[...]

    return pl.pallas_call(
        _inproj,
        grid=(NP_PAD // BR,),
        in_specs=[
            pl.BlockSpec((BR, EMB), lambda i: (i, 0)),
            pl.BlockSpec((EMB, EMB), lambda i: (0, 0)),
            pl.BlockSpec((EMB, 2), lambda i: (0, 0)),
        ],
        out_specs=[
            pl.BlockSpec((BR, EMB), lambda i: (i, 0)),
            pl.BlockSpec((BR, 2), lambda i: (i, 0)),
        ],
        out_shape=[
            jax.ShapeDtypeStruct((NP_PAD, EMB), jnp.float32),
            jax.ShapeDtypeStruct((NP_PAD, 2), jnp.float32),
        ],
    )(xp, win, aw0)


def _pack_h(h):
    h2 = jnp.concatenate([h[:, :HEMB], h[:, HEMB:]], axis=0)
    h2 = h2.reshape(-1, 2, 2, 16).transpose(0, 1, 3, 2).reshape(-1, HEMB)
    return h2.astype(jnp.bfloat16)


def _dual_gat(p_x, p_ei, p_ea, pp, v_x, v_ei, v_ea, pv):
    N = p_x.shape[0]
    srcp, dstp, e3p, eap, Wep, EPp = _prep_edges(p_ei, p_ea, pp)
    srcv, dstv, e3v, eav, Wev, EPv = _prep_edges(v_ei, v_ea, pv)
    nbp, nbv = EPp // (16 * BLK), EPv // (16 * BLK)

    src_all = jnp.concatenate([srcp, srcv]).reshape(-1, CHUNK)
    dst_all = jnp.concatenate([dstp, dstv]).reshape(-1, CHUNK)
    e3_all = jnp.concatenate([e3p, e3v])
    ea_all = jnp.concatenate([eap, eav])

    edge_pass = _edge_pass_kernel(nbp, nbv, EPp // CHUNK)

    hp, alp = _inproj_kernel(
        jnp.pad(p_x, ((0, NP_PAD - N), (0, 0))), pp['Win'],
        jnp.stack([pp['a1'][0], pp['a2'][0]], axis=1))
    hv, alv = _inproj_kernel(
        jnp.pad(v_x, ((0, NP_PAD - v_x.shape[0]), (0, 0))), pv['Win'],
        jnp.stack([pv['a1'][0], pv['a2'][0]], axis=1))
    h0pp, h0pv = hp, hv
    for l in range(L):
        e3m = e3_all[:, l].reshape(-1, CHUNK)
        h2 = jnp.concatenate([_pack_h(hp), _pack_h(hv)], axis=0)
        a1_all = jnp.concatenate([alp[:, 0], alv[:, 0]])
        a2_all = jnp.concatenate([alp[:, 1], alv[:, 1]])
        U, eas, dens = edge_pass(src_all, dst_all, e3m, ea_all, h2,
                                 a1_all, a2_all)
        beta = float(np.log(THETA / (l + 1) + 1.0))
        ln = min(l + 1, L - 1)
        hp, alp = _dense_layer(beta)(
            U[0, 0], U[0, 1], eas[0], dens[0], h0pp, Wep[l], pp['Wl'][l],
            jnp.stack([pp['a1'][ln], pp['a2'][ln]], axis=1))
        hv, alv = _dense_layer(beta)(
            U[1, 0], U[1, 1], eas[1], dens[1], h0pv, Wev[l], pv['Wl'][l],
            jnp.stack([pv['a1'][ln], pv['a2'][ln]], axis=1))
    return hp[:N], hv[:N]


def _seg_softmax(s, seg, n):
    m = jax.ops.segment_max(s, seg, num_segments=n)
    m = jnp.where(jnp.isfinite(m), m, 0.0)
    ex = jnp.exp(s - m[seg])
    den = jax.ops.segment_sum(ex, seg, num_segments=n)
    return ex / (den[seg] + 1e-16)


def _gap(x, batch, Wg, wv):
    sc = jnp.tanh(x @ Wg) @ wv
    a = _seg_softmax(sc, batch, G)
    return jax.ops.segment_sum(a[:, None] * x, batch, num_segments=G)


def _gru(x, h, p):
    gi = x @ p['Wih'] + p['bih']
    gh = h @ p['Whh'] + p['bhh']
    ir, iz, inn = jnp.split(gi, 3, axis=-1)
    hr, hz, hn = jnp.split(gh, 3, axis=-1)
    r = jax.nn.sigmoid(ir + hr)
    z = jax.nn.sigmoid(iz + hz)
    n = jnp.tanh(inn + r * hn)
    return (1.0 - z) * n + z * h


def _mlp(x, layers):
    for i, (W, b) in enumerate(layers):
        x = x @ W + b
        if i < len(layers) - 1:
            x = jax.nn.relu(x)
    return x


def kernel(p_x, p_edge_index, p_edge_attr, p_batch, v_x, v_edge_index,
           v_edge_attr, v_net_attrs, hidden, params):
    hp, hv = _dual_gat(p_x, p_edge_index, p_edge_attr, params['p_gnn'],
                       v_x, v_edge_index, v_edge_attr, params['v_gnn'])
    p = params
    batch_row = p_batch.astype(jnp.int32).reshape(1, -1)
    logits, h = _heads_kernel()(
        hp, hv, batch_row,
        p['p_gap']['Wg'], p['p_gap']['wv'].reshape(1, EMB),
        p['v_gap']['Wg'], p['v_gap']['wv'].reshape(1, EMB),
        v_net_attrs,
        p['attrs_mlp'][0][0], p['attrs_mlp'][1][0],
        p['fusion_mlp'][0][0], p['fusion_mlp'][0][1].reshape(1, -1),
        p['fusion_mlp'][1][0], p['fusion_mlp'][1][1].reshape(1, -1),
        p['fusion_mlp'][2][0], p['fusion_mlp'][2][1].reshape(1, -1),
        p['gru']['Wih'], p['gru']['Whh'],
        p['gru']['bih'].reshape(1, -1), p['gru']['bhh'].reshape(1, -1),
        p['actor'][0][0], p['actor'][0][1].reshape(1, -1),
        p['actor'][1][0], p['actor'][1][1].reshape(1, -1),
        p['actor'][2][0], p['actor'][2][1].reshape(1, -1),
        hidden)
    return (logits, h)


# speedup vs baseline: 1.2345x; 1.2345x over previous
import functools

import numpy as np

import jax
import jax.numpy as jnp
from jax import lax
from jax.experimental import pallas as pl
from jax.experimental.pallas import tpu as pltpu
from jax.experimental.pallas import tpu_sc as plsc

EMB = 128
HEMB = 64
L = 5
G = 64
ALPHA = 0.2
THETA = 0.2

NP_PAD = 10240
CHUNK = 128
KCH = 8
BLK = CHUNK * KCH
DEA = 16


def _edge_pass_kernel(nblocks):
    NPn = NP_PAD
    SL = NPn // 16
    mesh = plsc.VectorSubcoreMesh(core_axis_name="c", subcore_axis_name="s")

    @functools.partial(
        pl.kernel,
        out_type=(
            jax.ShapeDtypeStruct((2, NPn, HEMB), jnp.float32),
            jax.ShapeDtypeStruct((NPn, DEA), jnp.float32),
            jax.ShapeDtypeStruct((16, NPn), jnp.float32),
        ),
        mesh=mesh,
        compiler_params=pltpu.CompilerParams(
            needs_layout_passes=False, use_tc_tiling_on_sc=False),
        scratch_types=(
            pltpu.VMEM((NPn,), jnp.float32),
            pltpu.VMEM((NPn,), jnp.float32),
            pltpu.VMEM((NPn,), jnp.float32),
            pltpu.VMEM((KCH, CHUNK), jnp.int32),
            pltpu.VMEM((KCH, CHUNK), jnp.int32),
            pltpu.VMEM((KCH, CHUNK), jnp.int32),
            pltpu.VMEM((KCH, CHUNK), jnp.float32),
            pltpu.VMEM((KCH, CHUNK), jnp.float32),
            pltpu.VMEM((CHUNK, HEMB), jnp.bfloat16),
            pltpu.VMEM((CHUNK, HEMB), jnp.bfloat16),
            pltpu.VMEM((CHUNK, HEMB), jnp.float32),
            pltpu.VMEM((CHUNK, HEMB), jnp.float32),
            pltpu.VMEM((BLK, DEA), jnp.float32),
            pltpu.VMEM_SHARED((NPn, HEMB), jnp.float32),
            pltpu.VMEM_SHARED((NPn, DEA), jnp.float32),
            pltpu.SemaphoreType.DMA,
            pltpu.SemaphoreType.DMA,
            pltpu.SemaphoreType.DMA,
            pltpu.SemaphoreType.DMA,
            pltpu.SemaphoreType.DMA,
        ),
    )
    def edge_pass(src_hbm, dst_hbm, e3_hbm, ea_hbm, h2_hbm, a1_hbm, a2_hbm,
                  u_out, ea_out, den_out,
                  a1_v, a2_v, den_v, src_blk, dst_blk, gidx_blk, e3_blk,
                  ex_blk, rows_a, rows_b, sb_a, sb_b, ea_blk,
                  u_sh, easum_sh,
                  gsem_a, gsem_b, ssem_a, ssem_b, easem):
        c = lax.axis_index("c")
        s = lax.axis_index("s")
        srow = jnp.int32(c) * NPn
        rows = (rows_a, rows_b)
        sbufs = (sb_a, sb_b)
        gsem = (gsem_a, gsem_b)
        ssem = (ssem_a, ssem_b)

        pltpu.sync_copy(a1_hbm, a1_v)
        pltpu.sync_copy(a2_hbm, a2_v)

        z16 = jnp.zeros((16,), jnp.float32)

        def _zero_den(i, _):
            den_v[pl.ds(i * 16, 16)] = z16
            return 0
        lax.fori_loop(0, NPn // 16, _zero_den, 0)

        def _zero_rows(i, _):
            sb_a[i // 4, pl.ds((i % 4) * 16, 16)] = z16
            return 0
        lax.fori_loop(0, CHUNK * (HEMB // 16), _zero_rows, 0)

        def _zero_ea(i, _):
            ea_blk[i, :] = z16
            return 0
        lax.fori_loop(0, CHUNK, _zero_ea, 0)

        for k in range(SL // CHUNK):
            off = s * SL + k * CHUNK
            pltpu.sync_copy(sb_a, u_sh.at[pl.ds(off, CHUNK), :])
            pltpu.sync_copy(ea_blk.at[pl.ds(0, CHUNK), :],
                            easum_sh.at[pl.ds(off, CHUNK), :])
        plsc.subcore_barrier()

        def _block(b, _):
            blk0 = pl.multiple_of((s * nblocks + b) * KCH, KCH)
            pltpu.sync_copy(src_hbm.at[pl.ds(blk0, KCH), :], src_blk)
            pltpu.sync_copy(dst_hbm.at[pl.ds(blk0, KCH), :], dst_blk)
            pltpu.sync_copy(e3_hbm.at[pl.ds(blk0, KCH), :], e3_blk)
            pltpu.sync_copy(ea_hbm.at[pl.ds(blk0 * CHUNK, BLK), :], ea_blk)

            gdesc = [None] * KCH
            sdesc = [None] * KCH
            eadesc = [None] * KCH

            def _score(k):
                for g in range(CHUNK // 16):
                    sl16 = pl.ds(g * 16, 16)
                    s16 = src_blk[k, sl16]
                    d16 = dst_blk[k, sl16]
                    a1g = plsc.load_gather(a1_v, [s16])
                    a2g = plsc.load_gather(a2_v, [d16])
                    sc = a1g + a2g + e3_blk[k, sl16]
                    sc = jnp.where(sc >= 0.0, sc, 0.2 * sc)
                    ex = jnp.exp(sc)
                    ex_blk[k, sl16] = ex
                    plsc.addupdate_scatter(den_v, [d16], ex)
                    gidx_blk[k, sl16] = s16 + srow

            def _scale_scatter(k):
                gdesc[k].wait()
                if k >= 2:
                    sdesc[k - 2].wait()
                rbuf = rows[k % 2]
                sbuf = sbufs[k % 2]
                for g in range(CHUNK // 16):
                    ex16 = ex_blk[k, pl.ds(g * 16, 16)]
                    for i in range(16):
                        e = g * 16 + i
                        exs = ex16[i]
                        for j in range(HEMB // 32):
                            m32 = rbuf[e, pl.ds(j * 32, 32)]
                            pa, pb = plsc.unpack(
                                m32, format=plsc.PackFormat.INTERLEAVED,
                                preferred_element_type=jnp.float32)
                            sbuf[e, pl.ds(j * 32, 16)] = pa * exs
                            sbuf[e, pl.ds(j * 32 + 16, 16)] = pb * exs
                        eo = k * CHUNK + e
                        ea_blk[eo, :] = ea_blk[eo, :] * exs
                sdesc[k] = pltpu.async_copy(
                    sbuf, u_sh.at[dst_blk.at[k]], ssem[k % 2], add=True)
                eadesc[k] = pltpu.async_copy(
                    ea_blk.at[pl.ds(k * CHUNK, CHUNK), :],
                    easum_sh.at[dst_blk.at[k]], easem, add=True)

            for k in range(KCH):
                _score(k)
                if k >= 2:
                    _scale_scatter(k - 2)
                gdesc[k] = pltpu.async_copy(
                    h2_hbm.at[gidx_blk.at[k]], rows[k % 2], gsem[k % 2])
            for k in range(KCH - 2, KCH):
                _scale_scatter(k)
            for k in range(KCH - 2, KCH):
                sdesc[k].wait()
            for k in range(KCH):
                eadesc[k].wait()
            return 0
        lax.fori_loop(0, nblocks, _block, 0)

        @pl.when(c == 0)
        def _():
            pltpu.sync_copy(den_v, den_out.at[s])
        plsc.subcore_barrier()

        for k in range(SL // CHUNK):
            off = s * SL + k * CHUNK
            pltpu.sync_copy(u_sh.at[pl.ds(off, CHUNK), :],
                            u_out.at[c, pl.ds(off, CHUNK), :])

            @pl.when(c == 0)
            def _():
                pltpu.sync_copy(easum_sh.at[pl.ds(off, CHUNK), :],
                                ea_out.at[pl.ds(off, CHUNK), :])

    return edge_pass


BR = 512


def _dense_layer(beta):
    NPn = NP_PAD
    grid = (NPn // BR,)

    def body(u0, u1, eas, den16, h0, we, wl, aw, h_out, al_out):
        U = jnp.concatenate([u0[...], u1[...]], axis=1)
        den = jnp.sum(den16[...], axis=0)
        agg = (U + eas[...] @ we[...]) / (den + 1e-16)[:, None]
        sblk = (1.0 - ALPHA) * agg + ALPHA * h0[...]
        h = jax.nn.relu((1.0 - beta) * sblk + beta * (sblk @ wl[...]))
        h_out[...] = h
        al_out[...] = h @ aw[...]

    return pl.pallas_call(
        body,
        grid=grid,
        in_specs=[
            pl.BlockSpec((BR, HEMB), lambda i: (i, 0)),
            pl.BlockSpec((BR, HEMB), lambda i: (i, 0)),
            pl.BlockSpec((BR, DEA), lambda i: (i, 0)),
            pl.BlockSpec((16, BR), lambda i: (0, i)),
            pl.BlockSpec((BR, EMB), lambda i: (i, 0)),
            pl.BlockSpec((DEA, EMB), lambda i: (0, 0)),
            pl.BlockSpec((EMB, EMB), lambda i: (0, 0)),
            pl.BlockSpec((EMB, 2), lambda i: (0, 0)),
        ],
        out_specs=[
            pl.BlockSpec((BR, EMB), lambda i: (i, 0)),
            pl.BlockSpec((BR, 2), lambda i: (i, 0)),
        ],
        out_shape=[
            jax.ShapeDtypeStruct((NPn, EMB), jnp.float32),
            jax.ShapeDtypeStruct((NPn, 2), jnp.float32),
        ],
    )


def _heads_kernel():
    N = 10000

    def body(hp, hv, batch, wg_p, wv_p, wg_v, wv_v, va_in, aw1, aw2,
             fw1, fb1, fw2, fb2, fw3, fb3, wih, whh, bih, bhh,
             cw1, cb1, cw2, cb2, cw3, cb3, hid, logits_out, h_out):
        bi = jax.lax.broadcasted_iota(jnp.int32, (G, N), 0)
        B = (bi == batch[...]).astype(jnp.float32)

        def gap(x, wg, wv):
            t = jnp.tanh(x[...] @ wg[...])
            scgT = jax.lax.dot_general(wv[...], t,
                                       (((1,), (1,)), ((), ())))
            bs = jnp.where(B > 0, scgT, -1e30)
            m = jnp.max(bs, axis=1, keepdims=True)
            m = jnp.where(m < -9e29, 0.0, m)
            exn = jnp.where(B > 0, jnp.exp(scgT - m), 0.0)
            den = jnp.sum(exn, axis=1, keepdims=True)
            return (exn / (den + 1e-16)) @ x[...]

        pg = gap(hp, wg_p, wv_p)
        vg = gap(hv, wg_v, wv_v)
        va = jax.nn.relu(va_in[...] @ aw1[...]) @ aw2[...]
        fusion = jnp.concatenate([pg, vg, va], axis=-1)
        fe = jax.nn.relu(fusion @ fw1[...] + fb1[...])
        fe = jax.nn.relu(fe @ fw2[...] + fb2[...])
        fe = fe @ fw3[...] + fb3[...]
        gi = fe @ wih[...] + bih[...]
        gh = hid[...] @ whh[...] + bhh[...]
        ir, iz, inn = jnp.split(gi, 3, axis=-1)
        hr, hz, hn = jnp.split(gh, 3, axis=-1)
        r = jax.nn.sigmoid(ir + hr)
        z = jax.nn.sigmoid(iz + hz)
        n = jnp.tanh(inn + r * hn)
        h = (1.0 - z) * n + z * hid[...]
        lg = jax.nn.relu(h @ cw1[...] + cb1[...])
        lg = jax.nn.relu(lg @ cw2[...] + cb2[...])
        logits_out[...] = lg @ cw3[...] + cb3[...]
        h_out[...] = h

    return pl.pallas_call(
        body,
        out_shape=[
            jax.ShapeDtypeStruct((G, 100), jnp.float32),
            jax.ShapeDtypeStruct((G, EMB), jnp.float32),
        ],
    )


def _gat_sc(x, ei, ea_raw, p):
    N = x.shape[0]
    E = ei.shape[1]
    EPG = 16 * BLK
    EP = ((E + EPG - 1) // EPG) * EPG
    nblocks = EP // EPG

    src = jnp.concatenate([ei[0], jnp.zeros((EP - E,), ei.dtype)]).astype(jnp.int32)
    dst = jnp.concatenate([ei[1], jnp.zeros((EP - E,), ei.dtype)]).astype(jnp.int32)
    De = ea_raw.shape[1]
    ea = jnp.pad(ea_raw, ((0, EP - E), (0, DEA - De)))
    We = jnp.pad(p['We'], ((0, 0), (0, DEA - De), (0, 0)))

    c3 = jnp.einsum('lde,le->ld', We, p['a3'])
    e3 = ea @ c3.T
    e3 = jnp.where((jnp.arange(EP) < E)[:, None], e3, -1e30)

    srcm = src.reshape(EP // CHUNK, CHUNK)
    dstm = dst.reshape(EP // CHUNK, CHUNK)

    edge_pass = _edge_pass_kernel(nblocks)

    xp = jnp.pad(x, ((0, NP_PAD - N), (0, 0)))
    aw0 = jnp.stack([p['a1'][0], p['a2'][0]], axis=1)

    def _inproj(xr, win, aw, h_out, al_out):
        h = jax.nn.relu(xr[...] @ win[...])
        h_out[...] = h
        al_out[...] = h @ aw[...]

    h0p, alph = pl.pallas_call(
        _inproj,
        grid=(NP_PAD // BR,),
        in_specs=[
            pl.BlockSpec((BR, EMB), lambda i: (i, 0)),
            pl.BlockSpec((EMB, EMB), lambda i: (0, 0)),
            pl.BlockSpec((EMB, 2), lambda i: (0, 0)),
        ],
        out_specs=[
            pl.BlockSpec((BR, EMB), lambda i: (i, 0)),
            pl.BlockSpec((BR, 2), lambda i: (i, 0)),
        ],
        out_shape=[
            jax.ShapeDtypeStruct((NP_PAD, EMB), jnp.float32),
            jax.ShapeDtypeStruct((NP_PAD, 2), jnp.float32),
        ],
    )(xp, p['Win'], aw0)
    h = h0p
    for l in range(L):
        e3m = e3[:, l].reshape(EP // CHUNK, CHUNK)
        h2 = jnp.concatenate([h[:, :HEMB], h[:, HEMB:]], axis=0)
        h2 = h2.reshape(-1, 2, 2, 16).transpose(0, 1, 3, 2).reshape(-1, HEMB)
        h2 = h2.astype(jnp.bfloat16)
        U2, easum, den16 = edge_pass(srcm, dstm, e3m, ea, h2,
                                     alph[:, 0], alph[:, 1])
        beta = float(np.log(THETA / (l + 1) + 1.0))
        anext = p['a1'][min(l + 1, L - 1)], p['a2'][min(l + 1, L - 1)]
        aw = jnp.stack(anext, axis=1)
        h, alph = _dense_layer(beta)(
            U2[0], U2[1], easum, den16, h0p, We[l], p['Wl'][l], aw)
    return h[:N]


def _seg_softmax(s, seg, n):
    m = jax.ops.segment_max(s, seg, num_segments=n)
    m = jnp.where(jnp.isfinite(m), m, 0.0)
    ex = jnp.exp(s - m[seg])
    den = jax.ops.segment_sum(ex, seg, num_segments=n)
    return ex / (den[seg] + 1e-16)


def _gap(x, batch, Wg, wv):
    sc = jnp.tanh(x @ Wg) @ wv
    a = _seg_softmax(sc, batch, G)
    return jax.ops.segment_sum(a[:, None] * x, batch, num_segments=G)


def _gru(x, h, p):
    gi = x @ p['Wih'] + p['bih']
    gh = h @ p['Whh'] + p['bhh']
    ir, iz, inn = jnp.split(gi, 3, axis=-1)
    hr, hz, hn = jnp.split(gh, 3, axis=-1)
    r = jax.nn.sigmoid(ir + hr)
    z = jax.nn.sigmoid(iz + hz)
    n = jnp.tanh(inn + r * hn)
    return (1.0 - z) * n + z * h


def _mlp(x, layers):
    for i, (W, b) in enumerate(layers):
        x = x @ W + b
        if i < len(layers) - 1:
            x = jax.nn.relu(x)
    return x


def kernel(p_x, p_edge_index, p_edge_attr, p_batch, v_x, v_edge_index,
           v_edge_attr, v_net_attrs, hidden, params):
    hp = _gat_sc(p_x, p_edge_index, p_edge_attr, params['p_gnn'])
    hv = _gat_sc(v_x, v_edge_index, v_edge_attr, params['v_gnn'])
    p = params
    batch_row = p_batch.astype(jnp.int32).reshape(1, -1)
    logits, h = _heads_kernel()(
        hp, hv, batch_row,
        p['p_gap']['Wg'], p['p_gap']['wv'].reshape(1, EMB),
        p['v_gap']['Wg'], p['v_gap']['wv'].reshape(1, EMB),
        v_net_attrs,
        p['attrs_mlp'][0][0], p['attrs_mlp'][1][0],
        p['fusion_mlp'][0][0], p['fusion_mlp'][0][1].reshape(1, -1),
        p['fusion_mlp'][1][0], p['fusion_mlp'][1][1].reshape(1, -1),
        p['fusion_mlp'][2][0], p['fusion_mlp'][2][1].reshape(1, -1),
        p['gru']['Wih'], p['gru']['Whh'],
        p['gru']['bih'].reshape(1, -1), p['gru']['bhh'].reshape(1, -1),
        p['actor'][0][0], p['actor'][0][1].reshape(1, -1),
        p['actor'][1][0], p['actor'][1][1].reshape(1, -1),
        p['actor'][2][0], p['actor'][2][1].reshape(1, -1),
        hidden)
    return (logits, h)


# speedup vs baseline: 1.3358x; 1.0820x over previous
import functools

import numpy as np

import jax
import jax.numpy as jnp
from jax import lax
from jax.experimental import pallas as pl
from jax.experimental.pallas import tpu as pltpu
from jax.experimental.pallas import tpu_sc as plsc

EMB = 128
HEMB = 64
L = 5
G = 64
ALPHA = 0.2
THETA = 0.2

NP_PAD = 10240
CHUNK = 128
KCH = 8
BLK = CHUNK * KCH
DEA = 16


def _edge_pass_kernel(nblocks):
    NPn = NP_PAD
    SL = NPn // 16
    mesh = plsc.VectorSubcoreMesh(core_axis_name="c", subcore_axis_name="s")

    @functools.partial(
        pl.kernel,
        out_type=(
            jax.ShapeDtypeStruct((2, NPn, HEMB), jnp.float32),
            jax.ShapeDtypeStruct((NPn, DEA), jnp.float32),
            jax.ShapeDtypeStruct((16, NPn), jnp.float32),
        ),
        mesh=mesh,
        compiler_params=pltpu.CompilerParams(
            needs_layout_passes=False, use_tc_tiling_on_sc=False),
        scratch_types=(
            pltpu.VMEM((NPn,), jnp.float32),
            pltpu.VMEM((NPn,), jnp.float32),
            pltpu.VMEM((NPn,), jnp.float32),
            pltpu.VMEM((KCH, CHUNK), jnp.int32),
            pltpu.VMEM((KCH, CHUNK), jnp.int32),
            pltpu.VMEM((KCH, CHUNK), jnp.int32),
            pltpu.VMEM((KCH, CHUNK), jnp.float32),
            pltpu.VMEM((KCH, CHUNK), jnp.float32),
            pltpu.VMEM((CHUNK, HEMB), jnp.bfloat16),
            pltpu.VMEM((CHUNK, HEMB), jnp.bfloat16),
            pltpu.VMEM((CHUNK, HEMB), jnp.float32),
            pltpu.VMEM((CHUNK, HEMB), jnp.float32),
            pltpu.VMEM((BLK, DEA), jnp.float32),
            pltpu.VMEM_SHARED((NPn, HEMB), jnp.float32),
            pltpu.VMEM_SHARED((NPn, DEA), jnp.float32),
            pltpu.SemaphoreType.DMA,
            pltpu.SemaphoreType.DMA,
            pltpu.SemaphoreType.DMA,
            pltpu.SemaphoreType.DMA,
            pltpu.SemaphoreType.DMA,
        ),
    )
    def edge_pass(src_hbm, dst_hbm, e3_hbm, ea_hbm, h2_hbm, a1_hbm, a2_hbm,
                  u_out, ea_out, den_out,
                  a1_v, a2_v, den_v, src_blk, dst_blk, gidx_blk, e3_blk,
                  ex_blk, rows_a, rows_b, sb_a, sb_b, ea_blk,
                  u_sh, easum_sh,
                  gsem_a, gsem_b, ssem_a, ssem_b, easem):
        c = lax.axis_index("c")
        s = lax.axis_index("s")
        srow = jnp.int32(c) * NPn
        rows = (rows_a, rows_b)
        sbufs = (sb_a, sb_b)
        gsem = (gsem_a, gsem_b)
        ssem = (ssem_a, ssem_b)

        pltpu.sync_copy(a1_hbm, a1_v)
        pltpu.sync_copy(a2_hbm, a2_v)

        z16 = jnp.zeros((16,), jnp.float32)

        def _zero_den(i, _):
            den_v[pl.ds(i * 16, 16)] = z16
            return 0
        lax.fori_loop(0, NPn // 16, _zero_den, 0)

        def _zero_rows(i, _):
            sb_a[i // 4, pl.ds((i % 4) * 16, 16)] = z16
            return 0
        lax.fori_loop(0, CHUNK * (HEMB // 16), _zero_rows, 0)

        def _zero_ea(i, _):
            ea_blk[i, :] = z16
            return 0
        lax.fori_loop(0, CHUNK, _zero_ea, 0)

        for k in range(SL // CHUNK):
            off = s * SL + k * CHUNK
            pltpu.sync_copy(sb_a, u_sh.at[pl.ds(off, CHUNK), :])
            pltpu.sync_copy(ea_blk.at[pl.ds(0, CHUNK), :],
                            easum_sh.at[pl.ds(off, CHUNK), :])
        plsc.subcore_barrier()

        def _block(b, _):
            blk0 = pl.multiple_of((s * nblocks + b) * KCH, KCH)
            pltpu.sync_copy(src_hbm.at[pl.ds(blk0, KCH), :], src_blk)
            pltpu.sync_copy(dst_hbm.at[pl.ds(blk0, KCH), :], dst_blk)
            pltpu.sync_copy(e3_hbm.at[pl.ds(blk0, KCH), :], e3_blk)
            pltpu.sync_copy(ea_hbm.at[pl.ds(blk0 * CHUNK, BLK), :], ea_blk)

            gdesc = [None] * KCH
            sdesc = [None] * KCH
            eadesc = [None] * KCH

            def _score(k):
                for g in range(CHUNK // 16):
                    sl16 = pl.ds(g * 16, 16)
                    s16 = src_blk[k, sl16]
                    d16 = dst_blk[k, sl16]
                    a1g = plsc.load_gather(a1_v, [s16])
                    a2g = plsc.load_gather(a2_v, [d16])
                    sc = a1g + a2g + e3_blk[k, sl16]
                    sc = jnp.where(sc >= 0.0, sc, 0.2 * sc)
                    ex = jnp.exp(sc)
                    ex_blk[k, sl16] = ex
                    plsc.addupdate_scatter(den_v, [d16], ex)
                    gidx_blk[k, sl16] = s16 + srow

            def _scale_scatter(k):
                gdesc[k].wait()
                if k >= 2:
                    sdesc[k - 2].wait()
                rbuf = rows[k % 2]
                sbuf = sbufs[k % 2]
                for g in range(CHUNK // 16):
                    ex16 = ex_blk[k, pl.ds(g * 16, 16)]
                    for i in range(16):
                        e = g * 16 + i
                        exs = ex16[i]
                        for j in range(HEMB // 32):
                            m32 = rbuf[e, pl.ds(j * 32, 32)]
                            pa, pb = plsc.unpack(
                                m32, format=plsc.PackFormat.INTERLEAVED,
                                preferred_element_type=jnp.float32)
                            sbuf[e, pl.ds(j * 32, 16)] = pa * exs
                            sbuf[e, pl.ds(j * 32 + 16, 16)] = pb * exs
                        eo = k * CHUNK + e
                        ea_blk[eo, :] = ea_blk[eo, :] * exs
                sdesc[k] = pltpu.async_copy(
                    sbuf, u_sh.at[dst_blk.at[k]], ssem[k % 2], add=True)
                eadesc[k] = pltpu.async_copy(
                    ea_blk.at[pl.ds(k * CHUNK, CHUNK), :],
                    easum_sh.at[dst_blk.at[k]], easem, add=True)

            for k in range(KCH):
                _score(k)
                if k >= 2:
                    _scale_scatter(k - 2)
                gdesc[k] = pltpu.async_copy(
                    h2_hbm.at[gidx_blk.at[k]], rows[k % 2], gsem[k % 2])
            for k in range(KCH - 2, KCH):
                _scale_scatter(k)
            for k in range(KCH - 2, KCH):
                sdesc[k].wait()
            for k in range(KCH):
                eadesc[k].wait()
            return 0
        lax.fori_loop(0, nblocks, _block, 0)

        @pl.when(c == 0)
        def _():
            pltpu.sync_copy(den_v, den_out.at[s])
        plsc.subcore_barrier()

        for k in range(SL // CHUNK):
            off = s * SL + k * CHUNK
            pltpu.sync_copy(u_sh.at[pl.ds(off, CHUNK), :],
                            u_out.at[c, pl.ds(off, CHUNK), :])

            @pl.when(c == 0)
            def _():
                pltpu.sync_copy(easum_sh.at[pl.ds(off, CHUNK), :],
                                ea_out.at[pl.ds(off, CHUNK), :])

    return edge_pass


BR = 512


def _dense_layer(beta):
    NPn = NP_PAD
    grid = (NPn // BR,)

    def body(u0, u1, eas, den16, h0, we, wl, aw, h_out, al_out):
        U = jnp.concatenate([u0[...], u1[...]], axis=1)
        den = jnp.sum(den16[...], axis=0)
        agg = (U + eas[...] @ we[...]) / (den + 1e-16)[:, None]
        sblk = (1.0 - ALPHA) * agg + ALPHA * h0[...]
        h = jax.nn.relu((1.0 - beta) * sblk + beta * (sblk @ wl[...]))
        h_out[...] = h
        al_out[...] = h @ aw[...]

    return pl.pallas_call(
        body,
        grid=grid,
        in_specs=[
            pl.BlockSpec((BR, HEMB), lambda i: (i, 0)),
            pl.BlockSpec((BR, HEMB), lambda i: (i, 0)),
            pl.BlockSpec((BR, DEA), lambda i: (i, 0)),
            pl.BlockSpec((16, BR), lambda i: (0, i)),
            pl.BlockSpec((BR, EMB), lambda i: (i, 0)),
            pl.BlockSpec((DEA, EMB), lambda i: (0, 0)),
            pl.BlockSpec((EMB, EMB), lambda i: (0, 0)),
            pl.BlockSpec((EMB, 2), lambda i: (0, 0)),
        ],
        out_specs=[
            pl.BlockSpec((BR, EMB), lambda i: (i, 0)),
            pl.BlockSpec((BR, 2), lambda i: (i, 0)),
        ],
        out_shape=[
            jax.ShapeDtypeStruct((NPn, EMB), jnp.float32),
            jax.ShapeDtypeStruct((NPn, 2), jnp.float32),
        ],
    )


def _heads_kernel():
    N = 10000

    def body(hp, hv, batch, wg_p, wv_p, wg_v, wv_v, va_in, aw1, aw2,
             fw1, fb1, fw2, fb2, fw3, fb3, wih, whh, bih, bhh,
             cw1, cb1, cw2, cb2, cw3, cb3, hid, logits_out, h_out):
        bi = jax.lax.broadcasted_iota(jnp.int32, (G, N), 0)
        B = (bi == batch[...]).astype(jnp.float32)

        def gap(x, wg, wv):
            t = jnp.tanh(x[...] @ wg[...])
            scgT = jax.lax.dot_general(wv[...], t,
                                       (((1,), (1,)), ((), ())))
            bs = jnp.where(B > 0, scgT, -1e30)
            m = jnp.max(bs, axis=1, keepdims=True)
            m = jnp.where(m < -9e29, 0.0, m)
            exn = jnp.where(B > 0, jnp.exp(scgT - m), 0.0)
            den = jnp.sum(exn, axis=1, keepdims=True)
            return (exn / (den + 1e-16)) @ x[...]

        pg = gap(hp, wg_p, wv_p)
        vg = gap(hv, wg_v, wv_v)
        va = jax.nn.relu(va_in[...] @ aw1[...]) @ aw2[...]
        fusion = jnp.concatenate([pg, vg, va], axis=-1)
        fe = jax.nn.relu(fusion @ fw1[...] + fb1[...])
        fe = jax.nn.relu(fe @ fw2[...] + fb2[...])
        fe = fe @ fw3[...] + fb3[...]
        gi = fe @ wih[...] + bih[...]
        gh = hid[...] @ whh[...] + bhh[...]
        ir, iz, inn = jnp.split(gi, 3, axis=-1)
        hr, hz, hn = jnp.split(gh, 3, axis=-1)
        r = jax.nn.sigmoid(ir + hr)
        z = jax.nn.sigmoid(iz + hz)
        n = jnp.tanh(inn + r * hn)
        h = (1.0 - z) * n + z * hid[...]
        lg = jax.nn.relu(h @ cw1[...] + cb1[...])
        lg = jax.nn.relu(lg @ cw2[...] + cb2[...])
        logits_out[...] = lg @ cw3[...] + cb3[...]
        h_out[...] = h

    return pl.pallas_call(
        body,
        out_shape=[
            jax.ShapeDtypeStruct((G, 100), jnp.float32),
            jax.ShapeDtypeStruct((G, EMB), jnp.float32),
        ],
    )


def _gat_sc(x, ei, ea_raw, p):
    N = x.shape[0]
    E = ei.shape[1]
    EPG = 16 * BLK
    EP = ((E + EPG - 1) // EPG) * EPG
    nblocks = EP // EPG

    src = jnp.concatenate([ei[0], jnp.zeros((EP - E,), ei.dtype)]).astype(jnp.int32)
    dst = jnp.concatenate([ei[1], jnp.zeros((EP - E,), ei.dtype)]).astype(jnp.int32)
    De = ea_raw.shape[1]
    ea = jnp.pad(ea_raw, ((0, EP - E), (0, DEA - De)))
    We = jnp.pad(p['We'], ((0, 0), (0, DEA - De), (0, 0)))

    c3 = jnp.einsum('lde,le->ld', We, p['a3'])
    e3 = ea @ c3.T
    e3 = jnp.where((jnp.arange(EP) < E)[:, None], e3, -1e30)

    srcm = src.reshape(EP // CHUNK, CHUNK)
    dstm = dst.reshape(EP // CHUNK, CHUNK)

    edge_pass = _edge_pass_kernel(nblocks)

    xp = jnp.pad(x, ((0, NP_PAD - N), (0, 0)))
    aw0 = jnp.stack([p['a1'][0], p['a2'][0]], axis=1)

    def _inproj(xr, win, aw, h_out, al_out):
        h = jax.nn.relu(xr[...] @ win[...])
        h_out[...] = h
        al_out[...] = h @ aw[...]

    h0p, alph = pl.pallas_call(
        _inproj,
        grid=(NP_PAD // BR,),
        in_specs=[
            pl.BlockSpec((BR, EMB), lambda i: (i, 0)),
            pl.BlockSpec((EMB, EMB), lambda i: (0, 0)),
            pl.BlockSpec((EMB, 2), lambda i: (0, 0)),
        ],
        out_specs=[
            pl.BlockSpec((BR, EMB), lambda i: (i, 0)),
            pl.BlockSpec((BR, 2), lambda i: (i, 0)),
        ],
        out_shape=[
            jax.ShapeDtypeStruct((NP_PAD, EMB), jnp.float32),
            jax.ShapeDtypeStruct((NP_PAD, 2), jnp.float32),
        ],
    )(xp, p['Win'], aw0)
    h = h0p
    for l in range(L):
        e3m = e3[:, l].reshape(EP // CHUNK, CHUNK)
        h2 = jnp.concatenate([h[:, :HEMB], h[:, HEMB:]], axis=0)
        h2 = h2.reshape(-1, 2, 2, 16).transpose(0, 1, 3, 2).reshape(-1, HEMB)
        h2 = h2.astype(jnp.bfloat16)
        U2, easum, den16 = edge_pass(srcm, dstm, e3m, ea, h2,
                                     alph[:, 0], alph[:, 1])
        beta = float(np.log(THETA / (l + 1) + 1.0))
        anext = p['a1'][min(l + 1, L - 1)], p['a2'][min(l + 1, L - 1)]
        aw = jnp.stack(anext, axis=1)
        h, alph = _dense_layer(beta)(
            U2[0], U2[1], easum, den16, h0p, We[l], p['Wl'][l], aw)
    return h[:N]


def kernel(p_x, p_edge_index, p_edge_attr, p_batch, v_x, v_edge_index,
           v_edge_attr, v_net_attrs, hidden, params):
    hp = _gat_sc(p_x, p_edge_index, p_edge_attr, params['p_gnn'])
    hv = _gat_sc(v_x, v_edge_index, v_edge_attr, params['v_gnn'])
    p = params
    batch_row = p_batch.astype(jnp.int32).reshape(1, -1)
    logits, h = _heads_kernel()(
        hp, hv, batch_row,
        p['p_gap']['Wg'], p['p_gap']['wv'].reshape(1, EMB),
        p['v_gap']['Wg'], p['v_gap']['wv'].reshape(1, EMB),
        v_net_attrs,
        p['attrs_mlp'][0][0], p['attrs_mlp'][1][0],
        p['fusion_mlp'][0][0], p['fusion_mlp'][0][1].reshape(1, -1),
        p['fusion_mlp'][1][0], p['fusion_mlp'][1][1].reshape(1, -1),
        p['fusion_mlp'][2][0], p['fusion_mlp'][2][1].reshape(1, -1),
        p['gru']['Wih'], p['gru']['Whh'],
        p['gru']['bih'].reshape(1, -1), p['gru']['bhh'].reshape(1, -1),
        p['actor'][0][0], p['actor'][0][1].reshape(1, -1),
        p['actor'][1][0], p['actor'][1][1].reshape(1, -1),
        p['actor'][2][0], p['actor'][2][1].reshape(1, -1),
        hidden)
    return (logits, h)
